# baseline (speedup 1.0000x reference)
import functools

import jax
import jax.numpy as jnp
from jax import lax
from jax.experimental import pallas as pl
from jax.experimental.pallas import tpu as pltpu
from jax.experimental.pallas import tpu_sc as plsc

D_MODEL = 768
MAX_LEN = 4096
BATCH = 4
LANES = 16
NUM_CORES = 2
NUM_SUBCORES = 16
NUM_WORKERS = NUM_CORES * NUM_SUBCORES
ROWS_PER_WORKER = MAX_LEN // NUM_WORKERS
GROUPS_PER_BATCH = ROWS_PER_WORKER // LANES


SUB_ROWS = 32
NUM_SUB = ROWS_PER_WORKER // SUB_ROWS


def _body(sym_hbm, pe_hbm, out_hbm, pe_v, sym_v, zero_v, ssem, wsem, *rsems):
    wid = lax.axis_index("s") * NUM_CORES + lax.axis_index("c")
    base = wid * ROWS_PER_WORKER

    sym_copies = [
        pltpu.make_async_copy(
            sym_hbm.at[b, pl.ds(base, ROWS_PER_WORKER)],
            sym_v.at[pl.ds(b * ROWS_PER_WORKER, ROWS_PER_WORKER)],
            ssem,
        )
        for b in range(BATCH)
    ]
    for c in sym_copies:
        c.start()

    reads = []
    for k in range(NUM_SUB):
        c = pltpu.make_async_copy(
            pe_hbm.at[0, pl.ds(base + k * SUB_ROWS, SUB_ROWS)],
            pe_v.at[pl.ds(k * SUB_ROWS, SUB_ROWS)],
            rsems[k],
        )
        c.start()
        reads.append(c)

    writes = []
    for k in range(NUM_SUB):
        reads[k].wait()
        for b in range(BATCH):
            c = pltpu.make_async_copy(
                pe_v.at[pl.ds(k * SUB_ROWS, SUB_ROWS)],
                out_hbm.at[b, pl.ds(base + k * SUB_ROWS, SUB_ROWS)],
                wsem,
            )
            c.start()
            writes.append(c)

    zeros = jnp.zeros((LANES,), jnp.float32)
    for j in range(D_MODEL // LANES):
        zero_v[pl.ds(j * LANES, LANES)] = zeros

    for c in sym_copies:
        c.wait()
    for c in writes:
        c.wait()

    lane_iota = lax.iota(jnp.int32, LANES)
    for b in range(BATCH):
        def group_body(g, _, b=b):
            off = b * ROWS_PER_WORKER + g * LANES
            sv = sym_v[pl.ds(off, LANES)]
            pad = sv == 0
            n_pad = plsc.all_reduce_population_count(pad)[0]

            @pl.when(n_pad > 0)
            def _():
                def lane_body(i, _):
                    is_pad = plsc.all_reduce_population_count(
                        jnp.logical_and(pad, lane_iota == i))[0]

                    @pl.when(is_pad > 0)
                    def _():
                        row = base + g * LANES + i
                        pltpu.sync_copy(zero_v, out_hbm.at[b, row])

                    return 0

                lax.fori_loop(0, LANES, lane_body, 0)

            return 0

        lax.fori_loop(0, GROUPS_PER_BATCH, group_body, 0)


@functools.partial(
    pl.kernel,
    out_type=jax.ShapeDtypeStruct((BATCH, MAX_LEN, D_MODEL), jnp.float32),
    mesh=plsc.VectorSubcoreMesh(core_axis_name="c", subcore_axis_name="s"),
    compiler_params=pltpu.CompilerParams(
        needs_layout_passes=False,
        disable_bounds_checks=True,
        disable_semaphore_checks=True,
        skip_device_barrier=True,
    ),
    scratch_types=[
        pltpu.VMEM((ROWS_PER_WORKER, D_MODEL), jnp.float32),
        pltpu.VMEM((BATCH * ROWS_PER_WORKER,), jnp.int32),
        pltpu.VMEM((D_MODEL,), jnp.float32),
        pltpu.SemaphoreType.DMA,
        pltpu.SemaphoreType.DMA,
    ] + [pltpu.SemaphoreType.DMA] * NUM_SUB,
)
def _pe_broadcast(sym_hbm, pe_hbm, out_hbm, pe_v, sym_v, zero_v, ssem, wsem,
                  *rsems):
    _body(sym_hbm, pe_hbm, out_hbm, pe_v, sym_v, zero_v, ssem, wsem, *rsems)


def kernel(symbol, positional_encoding):
    sym = symbol.astype(jnp.int32)
    return _pe_broadcast(sym, positional_encoding)


# speedup vs baseline: 1.8476x; 1.8476x over previous
import jax
import jax.numpy as jnp
from jax.experimental import pallas as pl
from jax.experimental.pallas import tpu as pltpu

D_MODEL = 768
MAX_LEN = 4096
BATCH = 4
BLK = 512


def _tc_body(sym_ref, pe_ref, out_ref):
    m = (sym_ref[...] != 0).astype(jnp.float32)
    pe = pe_ref[...]
    out_ref[...] = pe[None, :, :] * m[:, :, None]


def kernel(symbol, positional_encoding):
    sym = symbol.astype(jnp.int32)
    pe = positional_encoding.reshape(MAX_LEN, D_MODEL)
    return pl.pallas_call(
        _tc_body,
        grid=(MAX_LEN // BLK,),
        in_specs=[
            pl.BlockSpec((BATCH, BLK), lambda i: (0, i)),
            pl.BlockSpec((BLK, D_MODEL), lambda i: (i, 0)),
        ],
        out_specs=pl.BlockSpec((BATCH, BLK, D_MODEL), lambda i: (0, i, 0)),
        out_shape=jax.ShapeDtypeStruct((BATCH, MAX_LEN, D_MODEL), jnp.float32),
        compiler_params=pltpu.CompilerParams(
            dimension_semantics=("arbitrary",),
        ),
    )(sym, pe)
